# baseline (speedup 1.0000x reference)
import functools

import jax
import jax.numpy as jnp
from jax import lax
from jax.experimental import pallas as pl
from jax.experimental.pallas import tpu as pltpu
from jax.experimental.pallas import tpu_sc as plsc

_W = 128
_L = 16


@functools.partial(jax.jit, static_argnums=(3, 4))
def _sc_gather(table_lo, table_hi, idx, NC, NW):
    B = idx.shape[0]
    D = table_lo.shape[1]
    HALF = table_lo.shape[0]
    bpw = B // NW
    K = bpw // _W
    mesh = plsc.VectorSubcoreMesh(core_axis_name="c", subcore_axis_name="s")

    @functools.partial(
        pl.kernel,
        mesh=mesh,
        compiler_params=pltpu.CompilerParams(use_tc_tiling_on_sc=False),
        out_type=jax.ShapeDtypeStruct((B + 8, D), jnp.float32),
        scratch_types=[
            pltpu.VMEM((bpw,), jnp.int32),
            pltpu.VMEM((K, _W), jnp.int32),
            pltpu.VMEM((K, _W), jnp.int32),
            pltpu.VMEM((K, _W), jnp.int32),
            pltpu.VMEM((K, _W), jnp.int32),
            pltpu.VMEM((bpw, D), jnp.float32),
            pltpu.VMEM((bpw, D), jnp.float32),
            pltpu.SemaphoreType.DMA,
        ],
    )
    def gather_kernel(tlo_hbm, thi_hbm, idx_hbm, out_hbm, idx_v,
                      lo_v, hi_v, plo_v, phi_v, rlo_v, rhi_v, sem):
        wid = lax.axis_index("s") * NC + lax.axis_index("c")
        base = wid * bpw
        pltpu.sync_copy(idx_hbm.at[pl.ds(base, bpw)], idx_v)

        lane = jnp.arange(_L, dtype=jnp.int32)
        trash = jnp.full((_L,), B, jnp.int32) + (lane & 7)
        for j in range(K):
            for m in range(_W // _L):
                off = j * _W + m * _L
                v = idx_v[pl.ds(off, _L)]
                is_hi = v >= HALF
                pos = base + off + lane
                lo_v[j, pl.ds(m * _L, _L)] = jnp.where(is_hi, 0, v)
                hi_v[j, pl.ds(m * _L, _L)] = jnp.where(is_hi, v - HALF, 0)
                plo_v[j, pl.ds(m * _L, _L)] = jnp.where(is_hi, trash, pos)
                phi_v[j, pl.ds(m * _L, _L)] = jnp.where(is_hi, pos, trash)
        plsc.subcore_barrier()

        gathers = []
        for j in range(K):
            gathers.append(pltpu.async_copy(
                tlo_hbm.at[lo_v.at[j]], rlo_v.at[pl.ds(j * _W, _W)], sem))
            gathers.append(pltpu.async_copy(
                thi_hbm.at[hi_v.at[j]], rhi_v.at[pl.ds(j * _W, _W)], sem))
        for g in gathers:
            g.wait()

        scat = []
        for j in range(K):
            scat.append(pltpu.async_copy(
                rlo_v.at[pl.ds(j * _W, _W)], out_hbm.at[plo_v.at[j]], sem))
            scat.append(pltpu.async_copy(
                rhi_v.at[pl.ds(j * _W, _W)], out_hbm.at[phi_v.at[j]], sem))
        for s in scat:
            s.wait()

    return gather_kernel(table_lo, table_hi, idx)


def kernel(batch, table):
    info = plsc.get_sparse_core_info()
    NW = info.num_cores * info.num_subcores
    idx = batch[0].astype(jnp.int32)
    half = table.shape[0] // 2
    out = _sc_gather(table[:half], table[half:], idx, info.num_cores, NW)
    return out[:batch.shape[1]]


# speedup vs baseline: 3.1318x; 3.1318x over previous
import functools

import jax
import jax.numpy as jnp
from jax import lax
from jax.experimental import pallas as pl
from jax.experimental.pallas import tpu as pltpu
from jax.experimental.pallas import tpu_sc as plsc

_C = 16


@functools.partial(jax.jit, static_argnums=(2, 3))
def _sc_gather(table, idx, NC, NW):
    B = idx.shape[0]
    D = table.shape[1]
    bpw = B // NW
    mesh = plsc.VectorSubcoreMesh(core_axis_name="c", subcore_axis_name="s")

    @functools.partial(
        pl.kernel,
        mesh=mesh,
        out_type=jax.ShapeDtypeStruct((B, D), jnp.float32),
        scratch_types=[
            pltpu.VMEM((bpw,), jnp.int32),
            pltpu.VMEM((bpw, D), jnp.float32),
            pltpu.SemaphoreType.DMA,
        ],
    )
    def gather_kernel(table_hbm, idx_hbm, out_hbm, idx_v, rows_v, sem):
        wid = lax.axis_index("s") * NC + lax.axis_index("c")
        base = wid * bpw
        pltpu.sync_copy(idx_hbm.at[pl.ds(base, bpw)], idx_v)

        def fire(g, carry):
            off = g * _C
            v16 = idx_v[pl.ds(off, _C)]
            for n in range(_C):
                pltpu.async_copy(table_hbm.at[pl.ds(v16[n], 1)],
                                 rows_v.at[pl.ds(off + n, 1)], sem)
            return carry

        lax.fori_loop(0, bpw // _C, fire, 0)

        def drain(g, carry):
            off = g * _C
            for n in range(_C):
                pltpu.make_async_copy(
                    table_hbm.at[pl.ds(0, 1)],
                    rows_v.at[pl.ds(off + n, 1)], sem).wait()
            return carry

        lax.fori_loop(0, bpw // _C, drain, 0)
        pltpu.sync_copy(rows_v, out_hbm.at[pl.ds(base, bpw)])

    return gather_kernel(table, idx)


def kernel(batch, table):
    info = plsc.get_sparse_core_info()
    NW = info.num_cores * info.num_subcores
    idx = batch[0].astype(jnp.int32)
    return _sc_gather(table, idx, info.num_cores, NW)
